# baseline (speedup 1.0000x reference)
import functools

import jax
import jax.numpy as jnp
from jax import lax
from jax.experimental import pallas as pl
from jax.experimental.pallas import tpu as pltpu
from jax.experimental.pallas import tpu_sc as plsc

N_TOKENS = 8192
D_MODEL = 4096
NUM_GROUPS = 16
BLOCK_N = 512

_LANES = 16
_NUM_WORKERS = 32
SC_TOKENS = 2048
_TC_TOKENS = N_TOKENS - SC_TOKENS
_TOK0 = _TC_TOKENS
_T_W = SC_TOKENS // _NUM_WORKERS
_SC_CHUNK = 8
_UNROLL = 8
_D_AUG = D_MODEL + 128


def _fused_kernel(x_ref, g_ref, w_ref, b_ref, o_ref):
    xb = x_ref[...]
    scores = lax.dot_general(
        xb, w_ref[...], (((1,), (1,)), ((), ())),
        preferred_element_type=jnp.float32)
    gid = g_ref[...]
    cols = lax.broadcasted_iota(jnp.int32, (xb.shape[0], NUM_GROUPS), 1)
    onehot = (cols == gid).astype(jnp.float32)
    o_ref[...] = jnp.sum((scores + b_ref[...]) * onehot, axis=1, keepdims=True)


def _tc_part(x_tc, g_tc, w2, b2):
    grid = _TC_TOKENS // BLOCK_N
    return pl.pallas_call(
        _fused_kernel,
        grid=(grid,),
        in_specs=[
            pl.BlockSpec((BLOCK_N, D_MODEL), lambda i: (i, 0)),
            pl.BlockSpec((BLOCK_N, 1), lambda i: (i, 0)),
            pl.BlockSpec((NUM_GROUPS, D_MODEL), lambda i: (0, 0)),
            pl.BlockSpec((1, NUM_GROUPS), lambda i: (0, 0)),
        ],
        out_specs=pl.BlockSpec((BLOCK_N, 1), lambda i: (i, 0)),
        out_shape=jax.ShapeDtypeStruct((_TC_TOKENS, 1), jnp.float32),
    )(x_tc, g_tc, w2, b2)


def _sc_part(x, group_sc, w_aug):
    mesh = plsc.VectorSubcoreMesh(core_axis_name="c", subcore_axis_name="s")
    steps = D_MODEL // (_LANES * _UNROLL)

    @functools.partial(
        pl.kernel, mesh=mesh,
        out_type=jax.ShapeDtypeStruct((SC_TOKENS,), jnp.float32),
        scratch_types=[
            pltpu.VMEM((_SC_CHUNK, D_MODEL), jnp.float32),
            pltpu.VMEM((_SC_CHUNK, _D_AUG), jnp.float32),
            pltpu.VMEM((_SC_CHUNK,), jnp.int32),
            pltpu.VMEM((_T_W,), jnp.float32),
            pltpu.VMEM((2 * _LANES,), jnp.float32),
            pltpu.SemaphoreType.DMA,
        ],
    )
    def dot_k(x_hbm, g_hbm, w_hbm, out_hbm, x_v, w_v, gidx, y_v, rot_v, sem):
        wid = lax.axis_index("s") * 2 + lax.axis_index("c")
        base = wid * _T_W
        lane = lax.broadcasted_iota(jnp.int32, (_LANES,), 0)
        acc16 = jnp.zeros((_LANES,), jnp.float32)
        for c in range(_T_W // _SC_CHUNK):
            tok = base + c * _SC_CHUNK
            pltpu.sync_copy(g_hbm.at[pl.ds(tok, _SC_CHUNK)], gidx)
            pltpu.sync_copy(x_hbm.at[pl.ds(_TOK0 + tok, _SC_CHUNK)], x_v)
            pltpu.async_copy(w_hbm.at[gidx], w_v, sem).wait()
            for t in range(_SC_CHUNK):
                ti = c * _SC_CHUNK + t

                def jstep(j, accs):
                    a0, a1, a2, a3 = accs
                    o = j * (_LANES * _UNROLL)
                    for k in range(_UNROLL):
                        off = o + k * _LANES
                        prod = (x_v[t, pl.ds(off, _LANES)]
                                * w_v[t, pl.ds(off, _LANES)])
                        if k % 4 == 0:
                            a0 = a0 + prod
                        elif k % 4 == 1:
                            a1 = a1 + prod
                        elif k % 4 == 2:
                            a2 = a2 + prod
                        else:
                            a3 = a3 + prod
                    return (a0, a1, a2, a3)

                z = jnp.zeros((_LANES,), jnp.float32)
                a = lax.fori_loop(0, steps, jstep, (z, z, z, z))
                v = (a[0] + a[1]) + (a[2] + a[3]) + w_v[t, pl.ds(D_MODEL,
                                                                 _LANES)]
                for sh in (8, 4, 2, 1):
                    rot_v[pl.ds(0, _LANES)] = v
                    rot_v[pl.ds(_LANES, _LANES)] = v
                    v = v + rot_v[pl.ds(sh, _LANES)]
                acc16 = acc16 + jnp.where(lane == ti % _LANES, v, 0.0)
                if ti % _LANES == _LANES - 1:
                    y_v[pl.ds((ti // _LANES) * _LANES, _LANES)] = acc16
                    acc16 = jnp.zeros((_LANES,), jnp.float32)
        pltpu.sync_copy(y_v, out_hbm.at[pl.ds(base, _T_W)])

    return dot_k(x, group_sc, w_aug)


def kernel(x, group, W, b):
    g1 = group.astype(jnp.int32)
    w2 = W.reshape(NUM_GROUPS, D_MODEL)
    b2 = b.reshape(1, NUM_GROUPS)
    w_aug = jnp.concatenate(
        [w2, b.reshape(NUM_GROUPS, 1),
         jnp.zeros((NUM_GROUPS, 127), jnp.float32)], axis=1)
    y_tc = _tc_part(x[:_TC_TOKENS], g1[:_TC_TOKENS].reshape(-1, 1), w2, b2)
    y_sc = _sc_part(x, g1[_TC_TOKENS:], w_aug)
    return jnp.concatenate([y_tc, y_sc.reshape(SC_TOKENS, 1)], axis=0)


# speedup vs baseline: 1.5058x; 1.5058x over previous
import functools

import jax
import jax.numpy as jnp
from jax import lax
from jax.experimental import pallas as pl
from jax.experimental.pallas import tpu as pltpu
from jax.experimental.pallas import tpu_sc as plsc

N_TOKENS = 8192
D_MODEL = 4096
NUM_GROUPS = 16
BLOCK_N = 512

_LANES = 16
_NUM_WORKERS = 32
SC_TOKENS = 2048
_TC_TOKENS = N_TOKENS - SC_TOKENS
_TOK0 = _TC_TOKENS
_T_W = SC_TOKENS // _NUM_WORKERS
_SC_CHUNK = 8
_UNROLL = 8
_D_AUG = D_MODEL + 128


def _fused_kernel(x_ref, g_ref, w_ref, b_ref, o_ref):
    xb = x_ref[...]
    scores = lax.dot_general(
        xb, w_ref[...], (((1,), (1,)), ((), ())),
        preferred_element_type=jnp.float32)
    gid = g_ref[...]
    cols = lax.broadcasted_iota(jnp.int32, (xb.shape[0], NUM_GROUPS), 1)
    onehot = (cols == gid).astype(jnp.float32)
    o_ref[...] = jnp.sum((scores + b_ref[...]) * onehot, axis=1, keepdims=True)


def _tc_part(x_tc, g_tc, w2, b2):
    grid = _TC_TOKENS // BLOCK_N
    return pl.pallas_call(
        _fused_kernel,
        grid=(grid,),
        in_specs=[
            pl.BlockSpec((BLOCK_N, D_MODEL), lambda i: (i, 0)),
            pl.BlockSpec((BLOCK_N, 1), lambda i: (i, 0)),
            pl.BlockSpec((NUM_GROUPS, D_MODEL), lambda i: (0, 0)),
            pl.BlockSpec((1, NUM_GROUPS), lambda i: (0, 0)),
        ],
        out_specs=pl.BlockSpec((BLOCK_N, 1), lambda i: (i, 0)),
        out_shape=jax.ShapeDtypeStruct((_TC_TOKENS, 1), jnp.float32),
    )(x_tc, g_tc, w2, b2)


def _sc_part(x, group_sc, w_aug):
    mesh = plsc.VectorSubcoreMesh(core_axis_name="c", subcore_axis_name="s")
    steps = D_MODEL // (_LANES * _UNROLL)

    @functools.partial(
        pl.kernel, mesh=mesh,
        out_type=jax.ShapeDtypeStruct((SC_TOKENS,), jnp.float32),
        scratch_types=[
            pltpu.VMEM((_SC_CHUNK, D_MODEL), jnp.float32),
            pltpu.VMEM((_SC_CHUNK, _D_AUG), jnp.float32),
            pltpu.VMEM((_SC_CHUNK,), jnp.int32),
            pltpu.VMEM((_T_W,), jnp.float32),
            pltpu.VMEM((2 * _LANES,), jnp.float32),
            pltpu.SemaphoreType.DMA,
        ],
    )
    def dot_k(x_hbm, g_hbm, w_hbm, out_hbm, x_v, w_v, gidx, y_v, rot_v, sem):
        wid = lax.axis_index("s") * 2 + lax.axis_index("c")
        base = wid * _T_W
        lane = lax.broadcasted_iota(jnp.int32, (_LANES,), 0)
        acc16 = jnp.zeros((_LANES,), jnp.float32)
        for c in range(_T_W // _SC_CHUNK):
            tok = base + c * _SC_CHUNK
            pltpu.sync_copy(g_hbm.at[pl.ds(tok, _SC_CHUNK)], gidx)
            pltpu.sync_copy(x_hbm.at[pl.ds(_TOK0 + tok, _SC_CHUNK)], x_v)
            pltpu.async_copy(w_hbm.at[gidx], w_v, sem).wait()
            for t in range(_SC_CHUNK):
                ti = c * _SC_CHUNK + t

                def jstep(j, accs):
                    a0, a1, a2, a3 = accs
                    o = j * (_LANES * _UNROLL)
                    for k in range(_UNROLL):
                        off = o + k * _LANES
                        prod = (x_v[t, pl.ds(off, _LANES)]
                                * w_v[t, pl.ds(off, _LANES)])
                        if k % 4 == 0:
                            a0 = a0 + prod
                        elif k % 4 == 1:
                            a1 = a1 + prod
                        elif k % 4 == 2:
                            a2 = a2 + prod
                        else:
                            a3 = a3 + prod
                    return (a0, a1, a2, a3)

                z = jnp.zeros((_LANES,), jnp.float32)
                a = lax.fori_loop(0, steps, jstep, (z, z, z, z))
                v = (a[0] + a[1]) + (a[2] + a[3]) + w_v[t, pl.ds(D_MODEL,
                                                                 _LANES)]
                for sh in (8, 4, 2, 1):
                    rot_v[pl.ds(0, _LANES)] = v
                    rot_v[pl.ds(_LANES, _LANES)] = v
                    v = v + rot_v[pl.ds(sh, _LANES)]
                acc16 = acc16 + jnp.where(lane == ti % _LANES, v, 0.0)
                if ti % _LANES == _LANES - 1:
                    y_v[pl.ds((ti // _LANES) * _LANES, _LANES)] = acc16
                    acc16 = jnp.zeros((_LANES,), jnp.float32)
        pltpu.sync_copy(y_v, out_hbm.at[pl.ds(base, _T_W)])

    return dot_k(x, group_sc, w_aug)


def kernel(x, group, W, b):
    g1 = group.astype(jnp.int32)
    w2 = W.reshape(NUM_GROUPS, D_MODEL)
    b2 = b.reshape(1, NUM_GROUPS)
    w_aug = jnp.concatenate(
        [w2, b.reshape(NUM_GROUPS, 1),
         jnp.zeros((NUM_GROUPS, 127), jnp.float32)], axis=1)
    y_tc = _tc_part(x, g1.reshape(-1, 1), w2, b2)
    y_sc = _sc_part(x, g1[_TC_TOKENS:], w_aug)
    return jnp.concatenate([y_tc, y_sc.reshape(SC_TOKENS, 1)], axis=0)
